# baseline (speedup 1.0000x reference)
import functools

import jax
import jax.numpy as jnp
from jax import lax
from jax.experimental import pallas as pl
from jax.experimental.pallas import tpu as pltpu
from jax.experimental.pallas import tpu_sc as plsc

_NPOS = 16384
_NNEG = 81920
_N = _NPOS + _NNEG
_D = 64
_L = 16
_NW = 32
_PW = _N // _NW
_C = 128
_NCH = _PW // _C
_HALF = 1000064
_TW = 1664


def _tc_pack2(ut, vt):
    def body(a_ref, b_ref, o_ref):
        at = jnp.swapaxes(a_ref[...], 0, 1)
        bt = jnp.swapaxes(b_ref[...], 0, 1)
        o_ref[...] = jnp.concatenate([at, bt], axis=1)[None]

    nb = _HALF // _TW

    def amap(t, i):
        return (0, jnp.where(t == 0, i, 0))

    def bmap(t, i):
        return (0, jnp.where(t == 0, nb + i, 0))

    def amap2(t, i):
        return (0, jnp.where(t == 1, i, 0))

    def bmap2(t, i):
        return (0, jnp.where(t == 1, nb + i, 0))

    def body2(ua_ref, ub_ref, va_ref, vb_ref, o_ref):
        t = pl.program_id(0)

        @pl.when(t == 0)
        def _u():
            at = jnp.swapaxes(ua_ref[...], 0, 1)
            bt = jnp.swapaxes(ub_ref[...], 0, 1)
            o_ref[...] = jnp.concatenate([at, bt], axis=1)[None]

        @pl.when(t == 1)
        def _v():
            at = jnp.swapaxes(va_ref[...], 0, 1)
            bt = jnp.swapaxes(vb_ref[...], 0, 1)
            o_ref[...] = jnp.concatenate([at, bt], axis=1)[None]

    return pl.pallas_call(
        body2,
        grid=(2, nb),
        in_specs=[
            pl.BlockSpec((_D, _TW), amap),
            pl.BlockSpec((_D, _TW), bmap),
            pl.BlockSpec((_D, _TW), amap2),
            pl.BlockSpec((_D, _TW), bmap2),
        ],
        out_specs=pl.BlockSpec((1, _TW, 2 * _D), lambda t, i: (t, i, 0)),
        out_shape=jax.ShapeDtypeStruct((2, _HALF, 2 * _D), jnp.float32),
    )(ut, ut, vt, vt)


def _sc_partials(u2, v2, u_idx, v_idx, paru, parv):
    mesh = plsc.VectorSubcoreMesh(core_axis_name="c", subcore_axis_name="s")

    @functools.partial(
        pl.kernel,
        mesh=mesh,
        compiler_params=pltpu.CompilerParams(use_tc_tiling_on_sc=False),
        out_type=jax.ShapeDtypeStruct((_N * _L,), jnp.float32),
        scratch_types=[
            pltpu.VMEM((_NCH, _C), jnp.int32),
            pltpu.VMEM((_NCH, _C), jnp.int32),
            pltpu.VMEM((_C, 2 * _D), jnp.float32),
            pltpu.VMEM((_C, 2 * _D), jnp.float32),
            pltpu.VMEM((_C * _L,), jnp.float32),
            pltpu.VMEM((_C * _L,), jnp.int32),
            pltpu.VMEM((_C * _L,), jnp.int32),
            pltpu.SemaphoreType.DMA,
        ],
    )
    def k(u_hbm, v_hbm, ui_hbm, vi_hbm, pu_hbm, pv_hbm, out_hbm,
          su_v, sv_v, ur_v, vr_v, part_v, pbu_v, pbv_v, sem):
        wid = lax.axis_index("s") * 2 + lax.axis_index("c")
        pltpu.sync_copy(ui_hbm.at[wid], su_v)
        pltpu.sync_copy(vi_hbm.at[wid], sv_v)

        def shift(j, carry):
            c = j // (_C // _L)
            sl = pl.ds((j % (_C // _L)) * _L, _L)
            iu = su_v[c, sl]
            iv = sv_v[c, sl]
            su_v[c, sl] = iu - jnp.where(iu >= _HALF, _HALF, 0)
            sv_v[c, sl] = iv - jnp.where(iv >= _HALF, _HALF, 0)
            return carry

        lax.fori_loop(0, _NCH * (_C // _L), shift, 0)

        for c in range(_NCH):
            cu = pltpu.async_copy(u_hbm.at[su_v.at[c]], ur_v, sem)
            cv = pltpu.async_copy(v_hbm.at[sv_v.at[c]], vr_v, sem)
            pltpu.sync_copy(pu_hbm.at[wid].at[c], pbu_v)
            pltpu.sync_copy(pv_hbm.at[wid].at[c], pbv_v)
            cu.wait()
            cv.wait()

            def row(i, carry):
                sl = pl.ds(i * _L, _L)
                mu = pbu_v[sl] == 1
                mv = pbv_v[sl] == 1
                acc = jnp.zeros((_L,), jnp.float32)
                for kk in range(_D // _L):
                    ulo = ur_v[i, pl.ds(kk * _L, _L)]
                    uhi = ur_v[i, pl.ds(_D + kk * _L, _L)]
                    vlo = vr_v[i, pl.ds(kk * _L, _L)]
                    vhi = vr_v[i, pl.ds(_D + kk * _L, _L)]
                    acc = acc + (jnp.where(mu, uhi, ulo)
                                 * jnp.where(mv, vhi, vlo))
                part_v[sl] = acc
                return carry

            lax.fori_loop(0, _C, row, 0)
            base = (wid * _PW + c * _C) * _L
            pltpu.sync_copy(part_v, out_hbm.at[pl.ds(base, _C * _L)])

    return k(u2, v2,
             u_idx.reshape(_NW, _NCH, _C), v_idx.reshape(_NW, _NCH, _C),
             paru.reshape(_NW, _NCH, _C * _L),
             parv.reshape(_NW, _NCH, _C * _L))


_BR = 1024


def _tc_loss(parts):
    p2 = parts.reshape(_N // 8, 128)

    def body(p_ref, o_ref):
        i = pl.program_id(0)

        @pl.when(i == 0)
        def _init():
            o_ref[0, 0] = jnp.float32(0.0)

        blk = p_ref[...]
        g = (lax.broadcasted_iota(jnp.int32, (8, 128), 1) // _L
             == lax.broadcasted_iota(jnp.int32, (8, 128), 0))
        gmat = g.astype(jnp.float32)
        st = lax.dot_general(gmat, blk, (((1,), (1,)), ((), ())),
                             preferred_element_type=jnp.float32)
        gid = ((i * _BR + lax.broadcasted_iota(jnp.int32, (8, _BR), 1)) * 8
               + lax.broadcasted_iota(jnp.int32, (8, _BR), 0))
        sgn = jnp.where(gid < _NPOS, jnp.float32(1.0), jnp.float32(-1.0))
        o_ref[0, 0] += jnp.sum(jax.nn.log_sigmoid(sgn * st))

    out = pl.pallas_call(
        body,
        grid=(_N // 8 // _BR,),
        in_specs=[pl.BlockSpec((_BR, 128), lambda i: (i, 0))],
        out_specs=pl.BlockSpec(memory_space=pltpu.MemorySpace.SMEM),
        out_shape=jax.ShapeDtypeStruct((1, 1), jnp.float32),
    )(p2)
    return out[0, 0]


def kernel(pos_u, pos_v, neg_u, neg_v, u_emb, v_emb):
    u_idx = jnp.concatenate([pos_u, neg_u]).astype(jnp.int32)
    v_idx = jnp.concatenate([pos_v, neg_v]).astype(jnp.int32)
    paru = jnp.broadcast_to((u_idx >= _HALF).astype(jnp.int32)[:, None],
                            (_N, _L))
    parv = jnp.broadcast_to((v_idx >= _HALF).astype(jnp.int32)[:, None],
                            (_N, _L))
    packed = _tc_pack2(u_emb.T, v_emb.T)
    parts = _sc_partials(packed[0], packed[1], u_idx, v_idx, paru, parv)
    return -_tc_loss(parts)


# speedup vs baseline: 1.4205x; 1.4205x over previous
import functools

import jax
import jax.numpy as jnp
from jax import lax
from jax.experimental import pallas as pl
from jax.experimental.pallas import tpu as pltpu
from jax.experimental.pallas import tpu_sc as plsc

_NPOS = 16384
_NNEG = 81920
_N = _NPOS + _NNEG
_D = 64
_L = 16
_NW = 32
_PW = _N // _NW
_C = 128
_NCH = _PW // _C
_HALF = 1000064
_TW = 1664


def _tc_pack(ut):
    def body(a_ref, b_ref, o_ref):
        ga = (lax.broadcasted_iota(jnp.int32, (_D, _D), 0)
              == lax.broadcasted_iota(jnp.int32, (_D, _D), 1))
        eye = ga.astype(jnp.float32)
        at = lax.dot_general(a_ref[...], eye, (((0,), (0,)), ((), ())),
                             preferred_element_type=jnp.float32)
        bt = lax.dot_general(b_ref[...], eye, (((0,), (0,)), ((), ())),
                             preferred_element_type=jnp.float32)
        o_ref[...] = jnp.concatenate([at, bt], axis=1)

    return pl.pallas_call(
        body,
        grid=(_HALF // _TW,),
        in_specs=[
            pl.BlockSpec((_D, _TW), lambda i: (0, i)),
            pl.BlockSpec((_D, _TW), lambda i: (0, _HALF // _TW + i)),
        ],
        out_specs=pl.BlockSpec((_TW, 2 * _D), lambda i: (i, 0)),
        out_shape=jax.ShapeDtypeStruct((_HALF, 2 * _D), jnp.float32),
    )(ut, ut)


def _sc_partials(u2, v2, u_idx, v_idx, paru, parv):
    mesh = plsc.VectorSubcoreMesh(core_axis_name="c", subcore_axis_name="s")

    @functools.partial(
        pl.kernel,
        mesh=mesh,
        compiler_params=pltpu.CompilerParams(use_tc_tiling_on_sc=False),
        out_type=jax.ShapeDtypeStruct((_N * _L,), jnp.float32),
        scratch_types=[
            pltpu.VMEM((_NCH, _C), jnp.int32),
            pltpu.VMEM((_NCH, _C), jnp.int32),
            pltpu.VMEM((_C, 2 * _D), jnp.float32),
            pltpu.VMEM((_C, 2 * _D), jnp.float32),
            pltpu.VMEM((_C * _L,), jnp.float32),
            pltpu.VMEM((_C * _L,), jnp.int32),
            pltpu.VMEM((_C * _L,), jnp.int32),
            pltpu.SemaphoreType.DMA,
        ],
    )
    def k(u_hbm, v_hbm, ui_hbm, vi_hbm, pu_hbm, pv_hbm, out_hbm,
          su_v, sv_v, ur_v, vr_v, part_v, pbu_v, pbv_v, sem):
        wid = lax.axis_index("s") * 2 + lax.axis_index("c")
        pltpu.sync_copy(ui_hbm.at[wid], su_v)
        pltpu.sync_copy(vi_hbm.at[wid], sv_v)

        def shift(j, carry):
            c = j // (_C // _L)
            sl = pl.ds((j % (_C // _L)) * _L, _L)
            iu = su_v[c, sl]
            iv = sv_v[c, sl]
            su_v[c, sl] = iu - jnp.where(iu >= _HALF, _HALF, 0)
            sv_v[c, sl] = iv - jnp.where(iv >= _HALF, _HALF, 0)
            return carry

        lax.fori_loop(0, _NCH * (_C // _L), shift, 0)

        for c in range(_NCH):
            cu = pltpu.async_copy(u_hbm.at[su_v.at[c]], ur_v, sem)
            cv = pltpu.async_copy(v_hbm.at[sv_v.at[c]], vr_v, sem)
            pltpu.sync_copy(pu_hbm.at[wid].at[c], pbu_v)
            pltpu.sync_copy(pv_hbm.at[wid].at[c], pbv_v)
            cu.wait()
            cv.wait()

            def row(i, carry):
                sl = pl.ds(i * _L, _L)
                mu = pbu_v[sl] == 1
                mv = pbv_v[sl] == 1
                acc = jnp.zeros((_L,), jnp.float32)
                for kk in range(_D // _L):
                    ulo = ur_v[i, pl.ds(kk * _L, _L)]
                    uhi = ur_v[i, pl.ds(_D + kk * _L, _L)]
                    vlo = vr_v[i, pl.ds(kk * _L, _L)]
                    vhi = vr_v[i, pl.ds(_D + kk * _L, _L)]
                    acc = acc + (jnp.where(mu, uhi, ulo)
                                 * jnp.where(mv, vhi, vlo))
                part_v[sl] = acc
                return carry

            lax.fori_loop(0, _C, row, 0)
            base = (wid * _PW + c * _C) * _L
            pltpu.sync_copy(part_v, out_hbm.at[pl.ds(base, _C * _L)])

    return k(u2, v2,
             u_idx.reshape(_NW, _NCH, _C), v_idx.reshape(_NW, _NCH, _C),
             paru.reshape(_NW, _NCH, _C * _L),
             parv.reshape(_NW, _NCH, _C * _L))


_BR = 1024


def _tc_loss(parts):
    p2 = parts.reshape(_N // 8, 128)

    def body(p_ref, o_ref):
        i = pl.program_id(0)

        @pl.when(i == 0)
        def _init():
            o_ref[0, 0] = jnp.float32(0.0)

        blk = p_ref[...]
        g = (lax.broadcasted_iota(jnp.int32, (8, 128), 1) // _L
             == lax.broadcasted_iota(jnp.int32, (8, 128), 0))
        gmat = g.astype(jnp.float32)
        st = lax.dot_general(gmat, blk, (((1,), (1,)), ((), ())),
                             preferred_element_type=jnp.float32)
        gid = ((i * _BR + lax.broadcasted_iota(jnp.int32, (8, _BR), 1)) * 8
               + lax.broadcasted_iota(jnp.int32, (8, _BR), 0))
        sgn = jnp.where(gid < _NPOS, jnp.float32(1.0), jnp.float32(-1.0))
        o_ref[0, 0] += jnp.sum(jax.nn.log_sigmoid(sgn * st))

    out = pl.pallas_call(
        body,
        grid=(_N // 8 // _BR,),
        in_specs=[pl.BlockSpec((_BR, 128), lambda i: (i, 0))],
        out_specs=pl.BlockSpec(memory_space=pltpu.MemorySpace.SMEM),
        out_shape=jax.ShapeDtypeStruct((1, 1), jnp.float32),
    )(p2)
    return out[0, 0]


def kernel(pos_u, pos_v, neg_u, neg_v, u_emb, v_emb):
    u_idx = jnp.concatenate([pos_u, neg_u]).astype(jnp.int32)
    v_idx = jnp.concatenate([pos_v, neg_v]).astype(jnp.int32)
    paru = jnp.broadcast_to((u_idx >= _HALF).astype(jnp.int32)[:, None],
                            (_N, _L))
    parv = jnp.broadcast_to((v_idx >= _HALF).astype(jnp.int32)[:, None],
                            (_N, _L))
    u2 = _tc_pack(u_emb.T)
    v2 = _tc_pack(v_emb.T)
    parts = _sc_partials(u2, v2, u_idx, v_idx, paru, parv)
    return -_tc_loss(parts)
